# Initial kernel scaffold; baseline (speedup 1.0000x reference)
import jax
import jax.numpy as jnp
from jax.experimental import pallas as pl


def kernel(x, w_xy, w_xh, w_hy, kh, ky):
    raise NotImplementedError("write your pallas kernel here")


# speedup vs baseline: 4.4606x; 4.4606x over previous
import functools

import jax
import jax.numpy as jnp
from jax.experimental import pallas as pl
from jax.experimental.pallas import tpu as pltpu

B = 128
N = 4096
TN = 512
NT = N // TN


def _mm_dual_body(x_ref, wa_ref, wb_ref, oa_ref, ob_ref):
    x = x_ref[...]
    oa_ref[...] = jnp.dot(x, wa_ref[...], preferred_element_type=jnp.float32)
    ob_ref[...] = jnp.dot(x, wb_ref[...], preferred_element_type=jnp.float32)


def _mm_dual(x, wa, wb):
    return pl.pallas_call(
        _mm_dual_body,
        grid=(NT,),
        in_specs=[
            pl.BlockSpec((B, N), lambda i: (0, 0)),
            pl.BlockSpec((N, TN), lambda i: (0, i)),
            pl.BlockSpec((N, TN), lambda i: (0, i)),
        ],
        out_specs=[
            pl.BlockSpec((B, TN), lambda i: (0, i)),
            pl.BlockSpec((B, TN), lambda i: (0, i)),
        ],
        out_shape=[
            jax.ShapeDtypeStruct((B, N), jnp.float32),
            jax.ShapeDtypeStruct((B, N), jnp.float32),
        ],
    )(x, wa, wb)


def _mm_sub_body(y0_ref, h_ref, w_ref, o_ref):
    hf = h_ref[...].astype(jnp.float32)
    o_ref[...] = y0_ref[...] - jnp.dot(hf, w_ref[...],
                                       preferred_element_type=jnp.float32)


def _mm_sub(y0, h, w):
    return pl.pallas_call(
        _mm_sub_body,
        grid=(NT,),
        in_specs=[
            pl.BlockSpec((B, TN), lambda i: (0, i)),
            pl.BlockSpec((B, N), lambda i: (0, 0)),
            pl.BlockSpec((N, TN), lambda i: (0, i)),
        ],
        out_specs=pl.BlockSpec((B, TN), lambda i: (0, i)),
        out_shape=jax.ShapeDtypeStruct((B, N), jnp.float32),
    )(y0, h, w)


def _kwta_body(k_ref, v_ref, o_ref):
    v = v_ref[...]
    s = jax.lax.bitcast_convert_type(v, jnp.int32)
    s = s ^ (jax.lax.shift_right_arithmetic(s, 31) & jnp.int32(0x7FFFFFFF))
    k = k_ref[0]

    def body(i, tu):
        bit = jnp.left_shift(jnp.int32(1), jnp.int32(31) - i)
        cand_u = tu | bit
        cand_s = cand_u ^ jnp.int32(-2147483648)
        cnt = jnp.sum((s >= cand_s).astype(jnp.int32), axis=1, keepdims=True)
        return jnp.where(cnt >= k, cand_u, tu)

    tu = jax.lax.fori_loop(0, 32, body, jnp.zeros((B, 1), jnp.int32))
    thr = tu ^ jnp.int32(-2147483648)
    o_ref[...] = (s >= thr).astype(jnp.int32)


def _kwta(v, k):
    karr = jnp.reshape(jnp.asarray(k, jnp.int32), (1,))
    return pl.pallas_call(
        _kwta_body,
        grid_spec=pltpu.PrefetchScalarGridSpec(
            num_scalar_prefetch=1,
            grid=(1,),
            in_specs=[pl.BlockSpec((B, N), lambda i, k: (0, 0))],
            out_specs=pl.BlockSpec((B, N), lambda i, k: (0, 0)),
        ),
        out_shape=jax.ShapeDtypeStruct((B, N), jnp.int32),
    )(karr, v)


def kernel(x, w_xy, w_xh, w_hy, kh, ky):
    hpre, y0 = _mm_dual(x, w_xh, w_xy)
    h = _kwta(hpre, kh)
    ypre = _mm_sub(y0, h, w_hy)
    y = _kwta(ypre, ky)
    return (h, y)


# speedup vs baseline: 4.5208x; 1.0135x over previous
import functools

import jax
import jax.numpy as jnp
from jax.experimental import pallas as pl
from jax.experimental.pallas import tpu as pltpu

B = 128
N = 4096
TK = 512
NK = N // TK


def _mm_dual_body(x_ref, wa_ref, wb_ref, oa_ref, ob_ref):
    pa = jnp.dot(x_ref[...], wa_ref[...], preferred_element_type=jnp.float32)
    pb = jnp.dot(x_ref[...], wb_ref[...], preferred_element_type=jnp.float32)

    @pl.when(pl.program_id(0) == 0)
    def _():
        oa_ref[...] = pa
        ob_ref[...] = pb

    @pl.when(pl.program_id(0) != 0)
    def _():
        oa_ref[...] += pa
        ob_ref[...] += pb


def _mm_dual(x, wa, wb):
    return pl.pallas_call(
        _mm_dual_body,
        grid=(NK,),
        in_specs=[
            pl.BlockSpec((B, TK), lambda k: (0, k)),
            pl.BlockSpec((TK, N), lambda k: (k, 0)),
            pl.BlockSpec((TK, N), lambda k: (k, 0)),
        ],
        out_specs=[
            pl.BlockSpec((B, N), lambda k: (0, 0)),
            pl.BlockSpec((B, N), lambda k: (0, 0)),
        ],
        out_shape=[
            jax.ShapeDtypeStruct((B, N), jnp.float32),
            jax.ShapeDtypeStruct((B, N), jnp.float32),
        ],
    )(x, wa, wb)


def _mm_sub_body(y0_ref, h_ref, w_ref, o_ref):
    hf = h_ref[...].astype(jnp.float32)
    p = jnp.dot(hf, w_ref[...], preferred_element_type=jnp.float32)

    @pl.when(pl.program_id(0) == 0)
    def _():
        o_ref[...] = y0_ref[...] - p

    @pl.when(pl.program_id(0) != 0)
    def _():
        o_ref[...] -= p


def _mm_sub(y0, h, w):
    return pl.pallas_call(
        _mm_sub_body,
        grid=(NK,),
        in_specs=[
            pl.BlockSpec((B, N), lambda k: (0, 0)),
            pl.BlockSpec((B, TK), lambda k: (0, k)),
            pl.BlockSpec((TK, N), lambda k: (k, 0)),
        ],
        out_specs=pl.BlockSpec((B, N), lambda k: (0, 0)),
        out_shape=jax.ShapeDtypeStruct((B, N), jnp.float32),
    )(y0, h, w)


def _kwta_body(k_ref, v_ref, o_ref):
    v = v_ref[...]
    s = jax.lax.bitcast_convert_type(v, jnp.int32)
    s = s ^ (jax.lax.shift_right_arithmetic(s, 31) & jnp.int32(0x7FFFFFFF))
    k = k_ref[0]

    def body(i, tu):
        bit = jnp.left_shift(jnp.int32(1), jnp.int32(31) - i)
        cand_u = tu | bit
        cand_s = cand_u ^ jnp.int32(-2147483648)
        cnt = jnp.sum((s >= cand_s).astype(jnp.int32), axis=1, keepdims=True)
        return jnp.where(cnt >= k, cand_u, tu)

    tu = jax.lax.fori_loop(0, 32, body, jnp.zeros((B, 1), jnp.int32))
    thr = tu ^ jnp.int32(-2147483648)
    o_ref[...] = (s >= thr).astype(jnp.int32)


def _kwta(v, k):
    karr = jnp.reshape(jnp.asarray(k, jnp.int32), (1,))
    return pl.pallas_call(
        _kwta_body,
        grid_spec=pltpu.PrefetchScalarGridSpec(
            num_scalar_prefetch=1,
            grid=(1,),
            in_specs=[pl.BlockSpec((B, N), lambda i, k: (0, 0))],
            out_specs=pl.BlockSpec((B, N), lambda i, k: (0, 0)),
        ),
        out_shape=jax.ShapeDtypeStruct((B, N), jnp.int32),
    )(karr, v)


def kernel(x, w_xy, w_xh, w_hy, kh, ky):
    hpre, y0 = _mm_dual(x, w_xh, w_xy)
    h = _kwta(hpre, kh)
    ypre = _mm_sub(y0, h, w_hy)
    y = _kwta(ypre, ky)
    return (h, y)
